# baseline (speedup 1.0000x reference)
import functools

import jax
import jax.numpy as jnp
from jax import lax
from jax.experimental import pallas as pl
from jax.experimental.pallas import tpu as pltpu
from jax.experimental.pallas import tpu_sc as plsc

E = 8
D = 768
F = 2048
S = 2048
TOPK = 2
DEG = (0.5, 0.25, 0.25)

TILE = 512
PAD = TOPK * S + E * TILE
NT = PAD // TILE
NW = 32
CH = (TOPK * S) // NW
TS = 512
NR = (TOPK * S) // 128

HC = CH // 2


@functools.cache
def _sc_kernels():
    mesh = plsc.VectorSubcoreMesh(core_axis_name="c", subcore_axis_name="s")

    @functools.partial(
        pl.kernel,
        out_type=jax.ShapeDtypeStruct((PAD, D), jnp.float32),
        mesh=mesh,
        scratch_types=[
            pltpu.VMEM((2, HC), jnp.int32),
            pltpu.VMEM((HC, D), jnp.float32),
            pltpu.VMEM((HC, D), jnp.float32),
            pltpu.SemaphoreType.DMA,
            pltpu.SemaphoreType.DMA,
            pltpu.SemaphoreType.DMA,
            pltpu.SemaphoreType.DMA,
        ],
    )
    def sc_dispatch(x_hbm, pos_hbm, xg_hbm, idx_v, rows_a, rows_b, s0, s1, s2, s3):
        wid = lax.axis_index("s") * 2 + lax.axis_index("c")
        src = (wid * CH) % S
        pltpu.sync_copy(pos_hbm.at[wid], idx_v)
        pltpu.sync_copy(x_hbm.at[pl.ds(src, HC)], rows_a)
        pltpu.sync_copy(x_hbm.at[pl.ds(src + HC, HC)], rows_b)
        sa = pltpu.async_copy(rows_a, xg_hbm.at[idx_v.at[0]], s2)
        sb = pltpu.async_copy(rows_b, xg_hbm.at[idx_v.at[1]], s3)
        sa.wait()
        sb.wait()
        del s0, s1

    @functools.partial(
        pl.kernel,
        out_type=jax.ShapeDtypeStruct((TOPK * S, D), jnp.float32),
        mesh=mesh,
        scratch_types=[
            pltpu.VMEM((2, HC), jnp.int32),
            pltpu.VMEM((HC, D), jnp.float32),
            pltpu.VMEM((HC, D), jnp.float32),
            pltpu.SemaphoreType.DMA,
            pltpu.SemaphoreType.DMA,
        ],
    )
    def sc_combine_gather(y_hbm, pos_hbm, yp_hbm, idx_v, rows_a, rows_b, s0, s1):
        wid = lax.axis_index("s") * 2 + lax.axis_index("c")
        base = wid * CH
        pltpu.sync_copy(pos_hbm.at[wid], idx_v)
        ga = pltpu.async_copy(y_hbm.at[idx_v.at[0]], rows_a, s0)
        gb = pltpu.async_copy(y_hbm.at[idx_v.at[1]], rows_b, s1)
        ga.wait()
        pltpu.sync_copy(rows_a, yp_hbm.at[pl.ds(base, HC)])
        gb.wait()
        pltpu.sync_copy(rows_b, yp_hbm.at[pl.ds(base + HC, HC)])

    return sc_dispatch, sc_combine_gather


def _route_body(e_ref, pos_ref, te_ref, used_ref):
    e2d = e_ref[...]

    ku = lax.broadcasted_iota(jnp.int32, (128, 128), 0)
    cu = lax.broadcasted_iota(jnp.int32, (128, 128), 1)
    triu = (ku <= cu).astype(jnp.float32)
    kr = lax.broadcasted_iota(jnp.int32, (NR, NR), 0)
    cr = lax.broadcasted_iota(jnp.int32, (NR, NR), 1)
    tril_strict = (kr > cr).astype(jnp.float32)

    tile_iota = lax.broadcasted_iota(jnp.int32, (1, NT), 1).astype(jnp.float32)
    pos = jnp.zeros((NR, 128), jnp.float32)
    te_acc = jnp.zeros((1, NT), jnp.float32)
    pad_off = jnp.zeros((1, 1), jnp.float32)
    for e in range(E):
        ohe = (e2d == e).astype(jnp.float32)
        csum = jnp.dot(ohe, triu, preferred_element_type=jnp.float32)
        tot = csum[:, 127:128]
        pref = jnp.dot(tril_strict, tot, preferred_element_type=jnp.float32)
        rank = csum - ohe + pref
        pos = pos + ohe * (rank + pad_off)
        cnt = pref[NR - 1 : NR, 0:1] + tot[NR - 1 : NR, 0:1]
        te_acc = te_acc + (tile_iota >= (pad_off / TILE)).astype(jnp.float32)
        pad_off = pad_off + jnp.floor((cnt + (TILE - 1)) / TILE) * TILE
    pos_ref[...] = pos.astype(jnp.int32)
    te_ref[...] = jnp.clip(te_acc - 1.0, 0.0, E - 1).astype(jnp.int32)
    used_ref[...] = (tile_iota * TILE < pad_off).astype(jnp.int32)


_route = pl.pallas_call(
    _route_body,
    in_specs=[
        pl.BlockSpec(memory_space=pltpu.VMEM),
    ],
    out_specs=[
        pl.BlockSpec(memory_space=pltpu.VMEM),
        pl.BlockSpec(memory_space=pltpu.VMEM),
        pl.BlockSpec(memory_space=pltpu.VMEM),
    ],
    out_shape=[
        jax.ShapeDtypeStruct((NR, 128), jnp.int32),
        jax.ShapeDtypeStruct((1, NT), jnp.int32),
        jax.ShapeDtypeStruct((1, NT), jnp.int32),
    ],
)


def _gemm_body(te_ref, used_ref, x_ref, wi_ref, wo_ref, y_ref, wi_bf, wo_bf):
    k = pl.program_id(0)

    @pl.when(used_ref[k] == 1)
    def _():
        changed = jnp.logical_or(
            k == 0, te_ref[k] != te_ref[jnp.maximum(k - 1, 0)]
        )

        @pl.when(changed)
        def _():
            wi_bf[...] = wi_ref[0].astype(jnp.bfloat16)
            wo_bf[...] = wo_ref[0].astype(jnp.bfloat16)

        h = jnp.maximum(
            jnp.dot(
                x_ref[...].astype(jnp.bfloat16),
                wi_bf[...],
                preferred_element_type=jnp.float32,
            ),
            0.0,
        ).astype(jnp.bfloat16)
        y_ref[...] = jnp.dot(h, wo_bf[...], preferred_element_type=jnp.float32)


_gemm = pl.pallas_call(
    _gemm_body,
    grid_spec=pltpu.PrefetchScalarGridSpec(
        num_scalar_prefetch=2,
        grid=(NT,),
        in_specs=[
            pl.BlockSpec((TILE, D), lambda k, te, used: (k, 0)),
            pl.BlockSpec((1, D, F), lambda k, te, used: (te[k], 0, 0)),
            pl.BlockSpec((1, F, D), lambda k, te, used: (te[k], 0, 0)),
        ],
        out_specs=pl.BlockSpec((TILE, D), lambda k, te, used: (k, 0)),
        scratch_shapes=[
            pltpu.VMEM((D, F), jnp.bfloat16),
            pltpu.VMEM((F, D), jnp.bfloat16),
        ],
    ),
    out_shape=jax.ShapeDtypeStruct((PAD, D), jnp.float32),
)


def _combine_body(y0_ref, y1_ref, er_ref, ent_ref, dfac_ref, x_ref, o_ref):
    er = er_ref[...]
    noise = jnp.zeros((TOPK, TS), jnp.float32)
    for e in range(E):
        noise = jnp.where(er == e, ent_ref[TOPK * e : TOPK * (e + 1), :], noise)
    slot_row = lax.broadcasted_iota(jnp.int32, (TOPK, TS), 0)
    wrow = (1.0 / E) * (1.0 + noise) * jnp.where(
        slot_row == 0, dfac_ref[0, 0], dfac_ref[0, 1]
    )
    wt = wrow.T
    s = y0_ref[...] * wt[:, 0:1] + y1_ref[...] * wt[:, 1:2]
    o_ref[...] = jnp.where(s != 0.0, s, x_ref[...])


_combine = pl.pallas_call(
    _combine_body,
    grid=(S // TS,),
    in_specs=[
        pl.BlockSpec((TS, D), lambda k: (k, 0)),
        pl.BlockSpec((TS, D), lambda k: (k + S // TS, 0)),
        pl.BlockSpec((TOPK, TS), lambda k: (0, k)),
        pl.BlockSpec((TOPK * E, TS), lambda k: (0, k)),
        pl.BlockSpec(memory_space=pltpu.SMEM),
        pl.BlockSpec((TS, D), lambda k: (k, 0)),
    ],
    out_specs=pl.BlockSpec((TS, D), lambda k: (k, 0)),
    out_shape=jax.ShapeDtypeStruct((S, D), jnp.float32),
)


def kernel(hidden_states, expert_index, wi, wo, exp_noise, topk):
    x = hidden_states.reshape(S, D)
    ei = expert_index.reshape(S, TOPK).astype(jnp.int32)
    e_row = ei.T
    e2d = e_row.reshape(NR, 128)
    ent = exp_noise[:, :, 0, :].reshape(E * TOPK, S)
    deg = jnp.asarray(DEG[:TOPK], dtype=jnp.float32)
    dfac = (deg * (jnp.arange(TOPK) < topk).astype(jnp.float32)).reshape(1, TOPK)

    pos2d, te, used = _route(e2d)

    sc_dispatch, sc_combine_gather = _sc_kernels()
    pos_w = pos2d.reshape(NW, 2, HC)
    xg = sc_dispatch(x, pos_w)
    y_sorted = _gemm(te.reshape(NT), used.reshape(NT), xg, wi, wo)
    y_pair = sc_combine_gather(y_sorted, pos_w)

    out = _combine(y_pair, y_pair, e_row, ent, dfac, x)
    return out.reshape(1, S, D)


# speedup vs baseline: 1.0543x; 1.0543x over previous
import functools

import jax
import jax.numpy as jnp
from jax import lax
from jax.experimental import pallas as pl
from jax.experimental.pallas import tpu as pltpu
from jax.experimental.pallas import tpu_sc as plsc

E = 8
D = 768
F = 2048
S = 2048
TOPK = 2
DEG = (0.5, 0.25, 0.25)

TILE = 512
PAD = TOPK * S + E * TILE
NT = PAD // TILE
NW = 32
CH = (TOPK * S) // NW
TS = 512
NR = (TOPK * S) // 128

HC = CH // 2


@functools.cache
def _sc_kernels():
    mesh = plsc.VectorSubcoreMesh(core_axis_name="c", subcore_axis_name="s")

    @functools.partial(
        pl.kernel,
        out_type=jax.ShapeDtypeStruct((PAD, D), jnp.float32),
        mesh=mesh,
        scratch_types=[
            pltpu.VMEM((CH,), jnp.int32),
            pltpu.VMEM((CH, D), jnp.float32),
            pltpu.SemaphoreType.DMA,
        ],
    )
    def sc_dispatch(x_hbm, pos_hbm, xg_hbm, idx_v, rows_v, sem):
        wid = lax.axis_index("s") * 2 + lax.axis_index("c")
        src = (wid * CH) % S
        pltpu.sync_copy(pos_hbm.at[wid], idx_v)
        pltpu.sync_copy(x_hbm.at[pl.ds(src, CH)], rows_v)
        pltpu.async_copy(rows_v, xg_hbm.at[idx_v], sem).wait()

    @functools.partial(
        pl.kernel,
        out_type=jax.ShapeDtypeStruct((TOPK * S, D), jnp.float32),
        mesh=mesh,
        scratch_types=[
            pltpu.VMEM((CH,), jnp.int32),
            pltpu.VMEM((CH, D), jnp.float32),
            pltpu.SemaphoreType.DMA,
        ],
    )
    def sc_combine_gather(y_hbm, pos_hbm, yp_hbm, idx_v, rows_v, sem):
        wid = lax.axis_index("s") * 2 + lax.axis_index("c")
        pltpu.sync_copy(pos_hbm.at[wid], idx_v)
        pltpu.async_copy(y_hbm.at[idx_v], rows_v, sem).wait()
        pltpu.sync_copy(rows_v, yp_hbm.at[pl.ds(wid * CH, CH)])

    return sc_dispatch, sc_combine_gather


def _route_body(e_ref, pos_ref, te_ref, used_ref):
    e2d = e_ref[...]

    ku = lax.broadcasted_iota(jnp.int32, (128, 128), 0)
    cu = lax.broadcasted_iota(jnp.int32, (128, 128), 1)
    triu = (ku <= cu).astype(jnp.float32)
    kr = lax.broadcasted_iota(jnp.int32, (NR, NR), 0)
    cr = lax.broadcasted_iota(jnp.int32, (NR, NR), 1)
    tril_strict = (kr > cr).astype(jnp.float32)

    tile_iota = lax.broadcasted_iota(jnp.int32, (1, NT), 1).astype(jnp.float32)
    pos = jnp.zeros((NR, 128), jnp.float32)
    te_acc = jnp.zeros((1, NT), jnp.float32)
    pad_off = jnp.zeros((1, 1), jnp.float32)
    for e in range(E):
        ohe = (e2d == e).astype(jnp.float32)
        csum = jnp.dot(ohe, triu, preferred_element_type=jnp.float32)
        tot = csum[:, 127:128]
        pref = jnp.dot(tril_strict, tot, preferred_element_type=jnp.float32)
        rank = csum - ohe + pref
        pos = pos + ohe * (rank + pad_off)
        cnt = pref[NR - 1 : NR, 0:1] + tot[NR - 1 : NR, 0:1]
        te_acc = te_acc + (tile_iota >= (pad_off / TILE)).astype(jnp.float32)
        pad_off = pad_off + jnp.floor((cnt + (TILE - 1)) / TILE) * TILE
    pos_ref[...] = pos.astype(jnp.int32)
    te_ref[...] = jnp.clip(te_acc - 1.0, 0.0, E - 1).astype(jnp.int32)
    used_ref[...] = (tile_iota * TILE < pad_off).astype(jnp.int32)


_route = pl.pallas_call(
    _route_body,
    in_specs=[
        pl.BlockSpec(memory_space=pltpu.VMEM),
    ],
    out_specs=[
        pl.BlockSpec(memory_space=pltpu.VMEM),
        pl.BlockSpec(memory_space=pltpu.VMEM),
        pl.BlockSpec(memory_space=pltpu.VMEM),
    ],
    out_shape=[
        jax.ShapeDtypeStruct((NR, 128), jnp.int32),
        jax.ShapeDtypeStruct((1, NT), jnp.int32),
        jax.ShapeDtypeStruct((1, NT), jnp.int32),
    ],
)


def _gemm_body(te_ref, used_ref, x_ref, wi_ref, wo_ref, y_ref):
    @pl.when(used_ref[pl.program_id(0)] == 1)
    def _():
        h = jnp.maximum(
            jnp.dot(
                x_ref[...].astype(jnp.bfloat16),
                wi_ref[0].astype(jnp.bfloat16),
                preferred_element_type=jnp.float32,
            ),
            0.0,
        ).astype(jnp.bfloat16)
        y_ref[...] = jnp.dot(
            h, wo_ref[0].astype(jnp.bfloat16), preferred_element_type=jnp.float32
        )


_gemm = pl.pallas_call(
    _gemm_body,
    grid_spec=pltpu.PrefetchScalarGridSpec(
        num_scalar_prefetch=2,
        grid=(NT,),
        in_specs=[
            pl.BlockSpec((TILE, D), lambda k, te, used: (k, 0)),
            pl.BlockSpec((1, D, F), lambda k, te, used: (te[k], 0, 0)),
            pl.BlockSpec((1, F, D), lambda k, te, used: (te[k], 0, 0)),
        ],
        out_specs=pl.BlockSpec((TILE, D), lambda k, te, used: (k, 0)),
    ),
    out_shape=jax.ShapeDtypeStruct((PAD, D), jnp.float32),
)


def _combine_body(y0_ref, y1_ref, er_ref, ent_ref, dfac_ref, x_ref, o_ref):
    er = er_ref[...]
    noise = jnp.zeros((TOPK, TS), jnp.float32)
    for e in range(E):
        noise = jnp.where(er == e, ent_ref[TOPK * e : TOPK * (e + 1), :], noise)
    slot_row = lax.broadcasted_iota(jnp.int32, (TOPK, TS), 0)
    wrow = (1.0 / E) * (1.0 + noise) * jnp.where(
        slot_row == 0, dfac_ref[0, 0], dfac_ref[0, 1]
    )
    wt = wrow.T
    s = y0_ref[...] * wt[:, 0:1] + y1_ref[...] * wt[:, 1:2]
    o_ref[...] = jnp.where(s != 0.0, s, x_ref[...])


_combine = pl.pallas_call(
    _combine_body,
    grid=(S // TS,),
    in_specs=[
        pl.BlockSpec((TS, D), lambda k: (k, 0)),
        pl.BlockSpec((TS, D), lambda k: (k + S // TS, 0)),
        pl.BlockSpec((TOPK, TS), lambda k: (0, k)),
        pl.BlockSpec((TOPK * E, TS), lambda k: (0, k)),
        pl.BlockSpec(memory_space=pltpu.SMEM),
        pl.BlockSpec((TS, D), lambda k: (k, 0)),
    ],
    out_specs=pl.BlockSpec((TS, D), lambda k: (k, 0)),
    out_shape=jax.ShapeDtypeStruct((S, D), jnp.float32),
)


def kernel(hidden_states, expert_index, wi, wo, exp_noise, topk):
    x = hidden_states.reshape(S, D)
    ei = expert_index.reshape(S, TOPK).astype(jnp.int32)
    e_row = ei.T
    e2d = e_row.reshape(NR, 128)
    ent = exp_noise[:, :, 0, :].reshape(E * TOPK, S)
    deg = jnp.asarray(DEG[:TOPK], dtype=jnp.float32)
    dfac = (deg * (jnp.arange(TOPK) < topk).astype(jnp.float32)).reshape(1, TOPK)

    pos2d, te, used = _route(e2d)

    sc_dispatch, sc_combine_gather = _sc_kernels()
    pos_w = pos2d.reshape(NW, CH)
    xg = sc_dispatch(x, pos_w)
    y_sorted = _gemm(te.reshape(NT), used.reshape(NT), xg, wi, wo)
    y_pair = sc_combine_gather(y_sorted, pos_w)

    out = _combine(y_pair, y_pair, e_row, ent, dfac, x)
    return out.reshape(1, S, D)
